# baseline (speedup 1.0000x reference)
import functools
import math

import jax
import jax.numpy as jnp
from jax import lax
from jax.experimental import pallas as pl
from jax.experimental.pallas import tpu as pltpu
from jax.experimental.pallas import tpu_sc as plsc

NUM_CORES = 2
NUM_SUBCORES = 16
NUM_WORKERS = NUM_CORES * NUM_SUBCORES
LANES = 16
CHUNK = 128


def _emb_body(x_hbm, lut_hbm, out_hbm, idx_v, rows_v, gsem):
    chunks_per_w, _ = idx_v.shape
    _, _, d = rows_v.shape
    scale = jnp.float32(math.sqrt(d))
    wid = lax.axis_index("s") * NUM_CORES + lax.axis_index("c")

    pltpu.sync_copy(x_hbm.at[wid], idx_v)

    def chunk_body(j, _):
        pltpu.async_copy(lut_hbm.at[idx_v.at[j]], rows_v.at[0], gsem).wait()

        def row_body(r, _):
            row = rows_v.at[0, r]
            for k in range(d // LANES):
                sl = pl.ds(k * LANES, LANES)
                row[sl] = row[sl] * scale
            return 0

        lax.fori_loop(0, CHUNK, row_body, 0, unroll=4)

        pltpu.sync_copy(rows_v.at[0], out_hbm.at[wid, j])
        return 0

    lax.fori_loop(0, chunks_per_w, chunk_body, 0)


def kernel(x, lut):
    b, h = x.shape
    v, d = lut.shape
    total = b * h
    assert total % (NUM_WORKERS * CHUNK) == 0
    assert d % LANES == 0
    chunks_per_w = total // (NUM_WORKERS * CHUNK)

    xg = x.astype(jnp.int32).reshape(NUM_WORKERS, chunks_per_w, CHUNK)

    mesh = plsc.VectorSubcoreMesh(core_axis_name="c", subcore_axis_name="s")
    run = pl.kernel(
        _emb_body,
        out_type=jax.ShapeDtypeStruct(
            (NUM_WORKERS, chunks_per_w, CHUNK, d), jnp.float32
        ),
        mesh=mesh,
        scratch_types=[
            pltpu.VMEM((chunks_per_w, CHUNK), jnp.int32),
            pltpu.VMEM((1, CHUNK, d), jnp.float32),
            pltpu.SemaphoreType.DMA,
        ],
        compiler_params=pltpu.CompilerParams(use_tc_tiling_on_sc=False),
    )
    out = run(xg, lut)
    return out.reshape(b, h, d)


# speedup vs baseline: 1.1580x; 1.1580x over previous
import math

import jax
import jax.numpy as jnp
from jax import lax
from jax.experimental import pallas as pl
from jax.experimental.pallas import tpu as pltpu
from jax.experimental.pallas import tpu_sc as plsc

NUM_CORES = 2
NUM_SUBCORES = 16
NUM_WORKERS = NUM_CORES * NUM_SUBCORES
LANES = 16
CHUNK = 128
K = 2
NSLOT = 4


def _emb_body(x_hbm, lut_hbm, out_hbm, idx_v, rows_v, g0, g1, g2, g3, s0,
              s1, s2, s3):
    chunks_per_w = idx_v.shape[0]
    d = rows_v.shape[-1]
    nsuper = chunks_per_w // K
    scale = jnp.float32(math.sqrt(d))
    wid = lax.axis_index("s") * NUM_CORES + lax.axis_index("c")
    gsem = (g0, g1, g2, g3)
    ssem = (s0, s1, s2, s3)

    def fire_gather(sup, slot):
        for b in range(K):
            pltpu.async_copy(
                lut_hbm.at[idx_v.at[sup * K + b]], rows_v.at[slot, b],
                gsem[slot])

    def drain_gather(slot):
        for b in range(K):
            pltpu.make_async_copy(
                lut_hbm.at[idx_v.at[0]], rows_v.at[slot, b],
                gsem[slot]).wait()

    def fire_store(sup, slot):
        for b in range(K):
            pltpu.async_copy(
                rows_v.at[slot, b], out_hbm.at[wid, sup * K + b], ssem[slot])

    def drain_store(slot):
        for b in range(K):
            pltpu.make_async_copy(
                rows_v.at[slot, b], out_hbm.at[wid, 0], ssem[slot]).wait()

    def scale_slot(slot):
        def row_body(r, _):
            for b in range(K):
                row = rows_v.at[slot, b, r]
                for k in range(d // LANES):
                    sl = pl.ds(k * LANES, LANES)
                    row[sl] = row[sl] * scale
            return 0

        lax.fori_loop(0, CHUNK, row_body, 0, unroll=2)

    pltpu.sync_copy(x_hbm.at[wid], idx_v)

    fire_gather(0, 0)
    fire_gather(1, 1)

    def group_body(t, _):
        for p in range(NSLOT):
            sup = t * NSLOT + p
            q = (p + 2) % NSLOT
            if p < 2:
                @pl.when(t >= 1)
                def _():
                    drain_store(q)
                fire_gather(sup + 2, q)
            else:
                drain_store(q)

                @pl.when(t < (nsuper // NSLOT) - 1)
                def _():
                    fire_gather(sup + 2, q)
            drain_gather(p)
            scale_slot(p)
            fire_store(sup, p)
        return 0

    lax.fori_loop(0, nsuper // NSLOT, group_body, 0)

    drain_store(2)
    drain_store(3)


def kernel(x, lut):
    b, h = x.shape
    v, d = lut.shape
    total = b * h
    assert total % (NUM_WORKERS * CHUNK * K * NSLOT) == 0
    assert d % LANES == 0
    chunks_per_w = total // (NUM_WORKERS * CHUNK)

    xg = x.astype(jnp.int32).reshape(NUM_WORKERS, chunks_per_w, CHUNK)

    mesh = plsc.VectorSubcoreMesh(core_axis_name="c", subcore_axis_name="s")
    run = pl.kernel(
        _emb_body,
        out_type=jax.ShapeDtypeStruct(
            (NUM_WORKERS, chunks_per_w, CHUNK, d), jnp.float32
        ),
        mesh=mesh,
        scratch_types=[
            pltpu.VMEM((chunks_per_w, CHUNK), jnp.int32),
            pltpu.VMEM((NSLOT, K, CHUNK, d), jnp.float32),
        ] + [pltpu.SemaphoreType.DMA] * 8,
        compiler_params=pltpu.CompilerParams(use_tc_tiling_on_sc=False),
    )
    out = run(xg, lut)
    return out.reshape(b, h, d)
